# baseline (speedup 1.0000x reference)
import functools

import jax
import jax.numpy as jnp
from jax.experimental import pallas as pl
from jax.experimental.pallas import tpu as pltpu

N = 10000
D_IN = 128
D_OUT = 128
CHUNK_ROWS = 200
NBUF = 4
NCHUNKS = N // CHUNK_ROWS


def _gcn_kernel(x_ref, a_ref, w_ref, o_ref, s_ref, buf_ref, sem_ref):
    i = pl.program_id(0)

    def chunk_copy(chunk_idx, slot):
        return pltpu.make_async_copy(
            a_ref.at[pl.ds(chunk_idx * CHUNK_ROWS, CHUNK_ROWS), :],
            buf_ref.at[slot],
            sem_ref.at[slot],
        )

    @pl.when(i == 0)
    def _bootstrap():
        s_ref[...] = jnp.dot(
            x_ref[...], w_ref[...], preferred_element_type=jnp.float32
        )
        for slot in range(NBUF):
            chunk_copy(slot, slot).start()

    slot = jax.lax.rem(i, NBUF)
    chunk_copy(i, slot).wait()
    o_ref[...] = jnp.dot(
        buf_ref[slot], s_ref[...], preferred_element_type=jnp.float32
    )

    @pl.when(i + NBUF < NCHUNKS)
    def _prefetch():
        chunk_copy(i + NBUF, slot).start()


@functools.partial(jax.jit, static_argnames=())
def kernel(X, A, W):
    n, d_in = X.shape
    d_out = W.shape[1]
    return pl.pallas_call(
        _gcn_kernel,
        grid=(NCHUNKS,),
        in_specs=[
            pl.BlockSpec((n, d_in), lambda i: (0, 0)),
            pl.BlockSpec(memory_space=pltpu.MemorySpace.HBM),
            pl.BlockSpec((d_in, d_out), lambda i: (0, 0)),
        ],
        out_specs=pl.BlockSpec((CHUNK_ROWS, d_out), lambda i: (i, 0)),
        out_shape=jax.ShapeDtypeStruct((n, d_out), jnp.float32),
        scratch_shapes=[
            pltpu.VMEM((n, d_out), jnp.float32),
            pltpu.VMEM((NBUF, CHUNK_ROWS, n), jnp.float32),
            pltpu.SemaphoreType.DMA((NBUF,)),
        ],
        compiler_params=pltpu.CompilerParams(
            vmem_limit_bytes=120 * 1024 * 1024,
        ),
    )(X, A, W)
